# baseline (speedup 1.0000x reference)
import jax
import jax.numpy as jnp
from jax.experimental import pallas as pl
from jax.experimental.pallas import tpu as pltpu

D_MODEL_ = 32
NUM_EXPERTS_ = 128
N_TOKENS_ = 8192
D_FF_ = 4 * D_MODEL_
BLK_ = 1024
NBLK_ = N_TOKENS_ // BLK_


def _moe_kernel(x_ref, gw_ref, gb_ref, ewt_ref, eb_ref, o_ref,
                mt_ref, b2_ref):
    i = pl.program_id(0)

    @pl.when(i == 0)
    def _build_m():
        xg = x_ref[:NUM_EXPERTS_, :]
        logits = jnp.dot(xg, gw_ref[...].T,
                         preferred_element_type=jnp.float32) + gb_ref[...]
        w = jax.nn.softmax(logits, axis=-1)
        mt_ref[...] = jnp.sum(ewt_ref[...] * w[None, :, :], axis=1)
        b2_ref[...] = jnp.sum(w * eb_ref[...], axis=0, keepdims=True)

    @pl.when(i > 0)
    def _matmul():
        o_ref[...] = jnp.dot(x_ref[...], mt_ref[...],
                             preferred_element_type=jnp.float32) + b2_ref[...]


def kernel(x, gate_w, gate_b, expert_w, expert_b):
    ewt = jnp.transpose(expert_w, (2, 0, 1))
    gb = gate_b.reshape(1, NUM_EXPERTS_)
    return pl.pallas_call(
        _moe_kernel,
        grid=(NBLK_ + 1,),
        in_specs=[
            pl.BlockSpec((BLK_, D_MODEL_),
                         lambda i: (jnp.maximum(i - 1, 0), 0)),
            pl.BlockSpec((NUM_EXPERTS_, D_MODEL_), lambda i: (0, 0)),
            pl.BlockSpec((1, NUM_EXPERTS_), lambda i: (0, 0)),
            pl.BlockSpec((D_MODEL_, NUM_EXPERTS_, NUM_EXPERTS_),
                         lambda i: (0, 0, 0)),
            pl.BlockSpec((NUM_EXPERTS_, D_FF_), lambda i: (0, 0)),
        ],
        out_specs=pl.BlockSpec((BLK_, NUM_EXPERTS_),
                               lambda i: (jnp.maximum(i - 1, 0), 0)),
        out_shape=jax.ShapeDtypeStruct((N_TOKENS_, NUM_EXPERTS_), jnp.float32),
        scratch_shapes=[
            pltpu.VMEM((D_MODEL_, NUM_EXPERTS_), jnp.float32),
            pltpu.VMEM((1, NUM_EXPERTS_), jnp.float32),
        ],
    )(x, gate_w, gb, ewt, expert_b)


# speedup vs baseline: 1.2258x; 1.2258x over previous
import jax
import jax.numpy as jnp
from jax.experimental import pallas as pl

D_MODEL_ = 32
NUM_EXPERTS_ = 128
N_TOKENS_ = 8192
D_FF_ = 4 * D_MODEL_


def _moe_kernel(x_ref, gw_ref, gb_ref, ewt_ref, eb_ref, o_ref):
    xg = x_ref[:NUM_EXPERTS_, :]
    logits = jnp.dot(xg, gw_ref[...].T,
                     preferred_element_type=jnp.float32) + gb_ref[...]
    w = jax.nn.softmax(logits, axis=-1)
    mt = jnp.sum(ewt_ref[...] * w[None, :, :], axis=1)
    b2 = jnp.sum(w * eb_ref[...], axis=0)
    o_ref[...] = jnp.dot(x_ref[...], mt,
                         preferred_element_type=jnp.float32) + b2[None, :]


def kernel(x, gate_w, gate_b, expert_w, expert_b):
    ewt = jnp.transpose(expert_w, (2, 0, 1))
    gb = gate_b.reshape(1, NUM_EXPERTS_)
    return pl.pallas_call(
        _moe_kernel,
        out_shape=jax.ShapeDtypeStruct((N_TOKENS_, NUM_EXPERTS_), jnp.float32),
    )(x, gate_w, gb, ewt, expert_b)
